# baseline (device time: 34994 ns/iter reference)
import jax
import jax.numpy as jnp
from jax import lax
from jax.experimental import pallas as pl
from jax.experimental.pallas import tpu as pltpu

SCALE = 64 ** -0.5
N_Y = 2


def _flash_decode(y_idx, Q3, K4, V4):
    b, h, d, skv = K4.shape
    bh = b // N_Y

    def body(y_ref, q_ref, k_ref, v_ref, out_ref,
             num_buf, rx_x, rx_y, rx_d, sems):
        i = pl.program_id(0)
        my_x = lax.axis_index("x")
        my_y = lax.axis_index("y")
        peers = (
            ((1 - my_x, my_y), rx_x),
            ((my_x, 1 - my_y), rx_y),
            ((1 - my_x, 1 - my_y), rx_d),
        )

        def rdma(p, rows, sem):
            (dev, dst), sl = peers[p], pl.ds(*rows)
            return pltpu.make_async_remote_copy(
                src_ref=num_buf.at[sl], dst_ref=dst.at[sl],
                send_sem=sems.at[sem], recv_sem=sems.at[sem + 1],
                device_id=dev, device_id_type=pl.DeviceIdType.MESH,
            )

        @pl.when(i == 0)
        def _():
            barrier = pltpu.get_barrier_semaphore()
            for (dev, _) in peers:
                pl.semaphore_signal(
                    barrier, inc=1, device_id=dev,
                    device_id_type=pl.DeviceIdType.MESH,
                )
            pl.semaphore_wait(barrier, 3)

        qt = jnp.transpose(q_ref[0], (1, 0))
        p_rows = []
        num_cols = []
        for hi in range(h):
            k = k_ref[0, hi]
            v = v_ref[0, hi]
            qcol = qt[:, hi:hi + 1]
            s = jnp.sum(k * qcol, axis=0, keepdims=True) * SCALE
            p = jnp.exp(s)
            p_rows.append(p)
            num_cols.append(jnp.sum(v * p, axis=1, keepdims=True))
        ps = jnp.concatenate(p_rows, axis=0)
        den_row = jnp.sum(ps, axis=1)[None]
        num_rows = jnp.concatenate(num_cols, axis=1)
        num_buf[i] = jnp.concatenate([num_rows, den_row], axis=0)

        @pl.when(i == bh - 2)
        def _():
            for p in range(3):
                rdma(p, (0, bh - 1), 4 * p).start()

        @pl.when(i == bh - 1)
        def _():
            for p in range(3):
                rdma(p, (bh - 1, 1), 4 * p + 2).start()
            for p in range(3):
                rdma(p, (0, bh - 1), 4 * p).wait()
                rdma(p, (bh - 1, 1), 4 * p + 2).wait()

            def norm_t(both):
                inv = 1.0 / both[:, d:d + 1, :]
                return jnp.transpose(both[:, :d, :] * inv, (0, 2, 1))

            out_ref[pl.ds(my_y * bh, bh)] = norm_t(num_buf[...] + rx_x[...])
            out_ref[pl.ds((1 - my_y) * bh, bh)] = norm_t(rx_y[...] + rx_d[...])

    half = lambda i, y_ref: (y_ref[0] * bh + i, 0, 0, 0)
    half3 = lambda i, y_ref: (y_ref[0] * bh + i, 0, 0)
    return pl.pallas_call(
        body,
        grid_spec=pltpu.PrefetchScalarGridSpec(
            num_scalar_prefetch=1,
            grid=(bh,),
            in_specs=[
                pl.BlockSpec((1, h, d), half3),
                pl.BlockSpec((1, h, d, skv), half),
                pl.BlockSpec((1, h, d, skv), half),
            ],
            out_specs=pl.BlockSpec(
                (b, h, d), lambda i, y_ref: (0, 0, 0)
            ),
            scratch_shapes=[
                pltpu.VMEM((bh, d + 1, h), jnp.float32),
                pltpu.VMEM((bh, d + 1, h), jnp.float32),
                pltpu.VMEM((bh, d + 1, h), jnp.float32),
                pltpu.VMEM((bh, d + 1, h), jnp.float32),
                pltpu.SemaphoreType.DMA((12,)),
            ],
        ),
        out_shape=jax.ShapeDtypeStruct((b, h, d), jnp.float32),
        compiler_params=pltpu.CompilerParams(
            collective_id=0,
            vmem_limit_bytes=100 * 1024 * 1024,
        ),
    )(y_idx, Q3, K4, V4)


def kernel(Q, K, V):
    b, _, h, d = Q.shape
    K4 = jnp.transpose(K, (0, 2, 3, 1))
    V4 = jnp.transpose(V, (0, 2, 3, 1))
    y_idx = lax.axis_index("y").reshape(1).astype(jnp.int32)
    out = _flash_decode(y_idx, Q.reshape(b, h, d), K4, V4)
    return out.reshape(b, 1, h, d)


# device time: 33530 ns/iter; 1.0437x vs baseline; 1.0437x over previous
import jax
import jax.numpy as jnp
from jax import lax
from jax.experimental import pallas as pl
from jax.experimental.pallas import tpu as pltpu

SCALE = 64 ** -0.5
N_Y = 2


def _flash_decode(y_idx, QT, K4, V4):
    b, h, d, skv = K4.shape
    bh = b // N_Y

    def body(y_ref, qt_ref, k_ref, v_ref, out_ref,
             num_buf, den_buf, nrx, drx, yrx,
             ns_send, ns_recv, ds_send, ds_recv, y_send, y_recv):
        i = pl.program_id(0)
        my_x = lax.axis_index("x")
        my_y = lax.axis_index("y")
        xnbr = (1 - my_x, my_y)
        ynbr = (my_x, 1 - my_y)

        @pl.when(i == 0)
        def _():
            barrier = pltpu.get_barrier_semaphore()
            for nbr in (xnbr, ynbr):
                pl.semaphore_signal(
                    barrier, inc=1, device_id=nbr,
                    device_id_type=pl.DeviceIdType.MESH,
                )
            pl.semaphore_wait(barrier, 2)

        qt = jnp.transpose(qt_ref[0], (1, 0))
        p_rows = []
        num_cols = []
        for hi in range(h):
            k = k_ref[0, hi]
            v = v_ref[0, hi]
            qcol = qt[:, hi:hi + 1]
            s = jnp.sum(k * qcol, axis=0, keepdims=True) * SCALE
            p = jnp.exp(s)
            p_rows.append(p)
            num_cols.append(jnp.sum(v * p, axis=1, keepdims=True))
        ps = jnp.concatenate(p_rows, axis=0)
        den_row = jnp.sum(ps, axis=1)[None]
        num_rows = jnp.concatenate(num_cols, axis=1)
        num_buf[i] = jnp.concatenate([num_rows, den_row], axis=0)

        def head_rdma():
            return pltpu.make_async_remote_copy(
                src_ref=num_buf.at[pl.ds(0, bh - 1)],
                dst_ref=nrx.at[pl.ds(0, bh - 1)],
                send_sem=ns_send, recv_sem=ns_recv,
                device_id=xnbr, device_id_type=pl.DeviceIdType.MESH,
            )

        @pl.when(i == bh - 2)
        def _():
            head_rdma().start()

        @pl.when(i == bh - 1)
        def _():
            r_tail = pltpu.make_async_remote_copy(
                src_ref=num_buf.at[pl.ds(bh - 1, 1)],
                dst_ref=nrx.at[pl.ds(bh - 1, 1)],
                send_sem=ds_send, recv_sem=ds_recv,
                device_id=xnbr, device_id_type=pl.DeviceIdType.MESH,
            )
            r_tail.start()
            head_rdma().wait()
            r_tail.wait()
            both = num_buf[...] + nrx[...]
            inv = 1.0 / both[:, d:d + 1, :]
            merged = jnp.transpose(both[:, :d, :] * inv, (0, 2, 1))
            out_ref[pl.ds(my_y * bh, bh)] = merged
            r_y = pltpu.make_async_remote_copy(
                src_ref=out_ref.at[pl.ds(my_y * bh, bh)],
                dst_ref=yrx,
                send_sem=y_send, recv_sem=y_recv,
                device_id=ynbr, device_id_type=pl.DeviceIdType.MESH,
            )
            r_y.start()
            r_y.wait()
            out_ref[pl.ds((1 - my_y) * bh, bh)] = yrx[...]

    half = lambda i, y_ref: (y_ref[0] * bh + i, 0, 0, 0)
    half3 = lambda i, y_ref: (y_ref[0] * bh + i, 0, 0)
    return pl.pallas_call(
        body,
        grid_spec=pltpu.PrefetchScalarGridSpec(
            num_scalar_prefetch=1,
            grid=(bh,),
            in_specs=[
                pl.BlockSpec((1, h, d), half3),
                pl.BlockSpec((1, h, d, skv), half),
                pl.BlockSpec((1, h, d, skv), half),
            ],
            out_specs=pl.BlockSpec(
                (b, h, d), lambda i, y_ref: (0, 0, 0)
            ),
            scratch_shapes=[
                pltpu.VMEM((bh, d + 1, h), jnp.float32),
                pltpu.VMEM((bh, 1, h), jnp.float32),
                pltpu.VMEM((bh, d + 1, h), jnp.float32),
                pltpu.VMEM((bh, 1, h), jnp.float32),
                pltpu.VMEM((bh, h, d), jnp.float32),
                pltpu.SemaphoreType.DMA,
                pltpu.SemaphoreType.DMA,
                pltpu.SemaphoreType.DMA,
                pltpu.SemaphoreType.DMA,
                pltpu.SemaphoreType.DMA,
                pltpu.SemaphoreType.DMA,
            ],
        ),
        out_shape=jax.ShapeDtypeStruct((b, h, d), jnp.float32),
        compiler_params=pltpu.CompilerParams(
            collective_id=0,
            vmem_limit_bytes=100 * 1024 * 1024,
        ),
    )(y_idx, QT, K4, V4)


def kernel(Q, K, V):
    b, _, h, d = Q.shape
    K4 = jnp.transpose(K, (0, 2, 3, 1))
    V4 = jnp.transpose(V, (0, 2, 3, 1))
    y_idx = lax.axis_index("y").reshape(1).astype(jnp.int32)
    out = _flash_decode(y_idx, Q.reshape(b, h, d), K4, V4)
    return out.reshape(b, 1, h, d)
